# Initial kernel scaffold; baseline (speedup 1.0000x reference)
import jax
import jax.numpy as jnp
from jax.experimental import pallas as pl


def kernel(x, edge_index, batch, edge_weight, W_rel1, b_rel1, W_root1, Wp, bp, W_rel3, b_rel3, W_root3, W1, b1, W2, b2):
    raise NotImplementedError("write your pallas kernel here")


# speedup vs baseline: 8.3946x; 8.3946x over previous
import functools
import math

import jax
import jax.numpy as jnp
from jax.experimental import pallas as pl

N = 10000
E = 320000
F_IN = 128
H = 512
C = 2
B = 8
OUT = 10
EPS = 1e-15
BLK = 1000
NB = N // BLK

_INTERPRET = False


def _dd0(a, b):
    return jax.lax.dot_general(a, b, (((0,), (0,)), ((), ())),
                               preferred_element_type=jnp.float32)


def _mm(a, b):
    return jnp.dot(a, b, preferred_element_type=jnp.float32)


def _tca_body(agg_ref, x_ref, wrel_ref, brel_ref, wroot_ref, wp_ref, bp_ref,
              h_ref, st_ref, sl_ref):
    agg = jnp.sum(agg_ref[...], axis=0)
    h = _mm(agg, wrel_ref[...]) + _mm(x_ref[...], wroot_ref[...]) + brel_ref[...]
    h = jnp.maximum(h, 0.0)
    sl = _mm(h, wp_ref[...]) + bp_ref[...]
    m = jnp.max(sl, axis=1, keepdims=True)
    e = jnp.exp(sl - m)
    s = e / jnp.sum(e, axis=1, keepdims=True)
    h_ref[...] = h
    sl_ref[...] = sl
    st_ref[...] = jnp.concatenate([s] * 8, axis=1)


def _tc_a(agg_parts, x, W_rel1, b_rel1, W_root1, Wp, bp):
    K = agg_parts.shape[0]
    return pl.pallas_call(
        _tca_body,
        grid=(NB,),
        in_specs=[
            pl.BlockSpec((K, BLK, F_IN), lambda i: (0, i, 0)),
            pl.BlockSpec((BLK, F_IN), lambda i: (i, 0)),
            pl.BlockSpec((F_IN, H), lambda i: (0, 0)),
            pl.BlockSpec((1, H), lambda i: (0, 0)),
            pl.BlockSpec((F_IN, H), lambda i: (0, 0)),
            pl.BlockSpec((H, C), lambda i: (0, 0)),
            pl.BlockSpec((1, C), lambda i: (0, 0)),
        ],
        out_specs=[
            pl.BlockSpec((BLK, H), lambda i: (i, 0)),
            pl.BlockSpec((BLK, 16), lambda i: (i, 0)),
            pl.BlockSpec((BLK, C), lambda i: (i, 0)),
        ],
        out_shape=[
            jax.ShapeDtypeStruct((N, H), jnp.float32),
            jax.ShapeDtypeStruct((N, 16), jnp.float32),
            jax.ShapeDtypeStruct((N, C), jnp.float32),
        ],
        interpret=_INTERPRET,
    )(agg_parts, x, W_rel1, b_rel1.reshape(1, H), W_root1, Wp, bp.reshape(1, C))


def _tcb_body(h_ref, st_ref, oh2_ref, q_ref, deg_ref, accO_ref, accM_ref):
    i = pl.program_id(0)
    P = oh2_ref[...] * st_ref[...]
    s = st_ref[...][:, 0:C]
    q = jnp.sum(q_ref[...], axis=0)
    deg = jnp.sum(deg_ref[...], axis=0)
    o16 = _dd0(P, h_ref[...])
    ss16 = _dd0(P, s)
    adj16 = _dd0(P, q)
    degssq = deg * jnp.sum(s * s, axis=1, keepdims=True)
    den16 = _dd0(oh2_ref[...], degssq)

    @pl.when(i == 0)
    def _():
        accO_ref[...] = jnp.zeros_like(accO_ref)
        accM_ref[...] = jnp.zeros_like(accM_ref)

    accO_ref[...] += o16
    accM_ref[:, 0:2] += ss16
    accM_ref[:, 2:4] += adj16
    accM_ref[:, 4:5] += den16


def _tc_b(h, st, oh2, q_parts, deg_parts):
    Kq = q_parts.shape[0]
    Kd = deg_parts.shape[0]
    return pl.pallas_call(
        _tcb_body,
        grid=(NB,),
        in_specs=[
            pl.BlockSpec((BLK, H), lambda i: (i, 0)),
            pl.BlockSpec((BLK, 16), lambda i: (i, 0)),
            pl.BlockSpec((BLK, 16), lambda i: (i, 0)),
            pl.BlockSpec((Kq, BLK, C), lambda i: (0, i, 0)),
            pl.BlockSpec((Kd, BLK, 1), lambda i: (0, i, 0)),
        ],
        out_specs=[
            pl.BlockSpec((16, H), lambda i: (0, 0)),
            pl.BlockSpec((16, 128), lambda i: (0, 0)),
        ],
        out_shape=[
            jax.ShapeDtypeStruct((16, H), jnp.float32),
            jax.ShapeDtypeStruct((16, 128), jnp.float32),
        ],
        interpret=_INTERPRET,
    )(h, st, oh2, q_parts, deg_parts)


def _tcc_body(accO_ref, accM_ref, wr3_ref, br3_ref, wo3_ref, w1_ref, b1_ref,
              w2_ref, b2_ref, logp_ref, mc_ref, ol_ref, adj_ref):
    accO = accO_ref[...]
    accM = accM_ref[...]
    ss16 = accM[:, 0:2]
    adj16 = accM[:, 2:4]
    den16 = accM[:, 4:5]

    ri16 = jax.lax.broadcasted_iota(jnp.int32, (16, 16), 0)
    ci16 = jax.lax.broadcasted_iota(jnp.int32, (16, 16), 1)
    I16 = (ri16 == ci16).astype(jnp.float32)
    blockmask = (ri16 // 2 == ci16 // 2).astype(jnp.float32)
    G0 = (ci16 == 2 * (ri16 // 2)).astype(jnp.float32)
    G1 = (ci16 == 2 * (ri16 // 2) + 1).astype(jnp.float32)
    ri82 = jax.lax.broadcasted_iota(jnp.int32, (8, 16), 0)
    ci82 = jax.lax.broadcasted_iota(jnp.int32, (8, 16), 1)
    Pair8 = (ci82 // 2 == ri82).astype(jnp.float32)
    Geven = (ci82 == 2 * ri82).astype(jnp.float32)
    ri168 = jax.lax.broadcasted_iota(jnp.int32, (16, 8), 0)
    ci168 = jax.lax.broadcasted_iota(jnp.int32, (16, 8), 1)
    G2 = (ri168 // 2 == ci168).astype(jnp.float32)
    ri2 = jax.lax.broadcasted_iota(jnp.int32, (16, 2), 0)
    ci2 = jax.lax.broadcasted_iota(jnp.int32, (16, 2), 1)
    diagmask = (ci2 == ri2 % 2).astype(jnp.float32)

    diag16 = jnp.sum(adj16 * diagmask, axis=1, keepdims=True)
    num8 = _mm(Pair8, diag16)
    den8 = _mm(Geven, den16)
    mc = -jnp.mean(num8 / (den8 + EPS))
    mc_ref[...] = jnp.full((1, 1), 0.0) + mc

    rs = jnp.sum(ss16 * ss16, axis=1, keepdims=True)
    nb8 = jnp.sqrt(_mm(Pair8, rs))
    nb16 = _mm(G2, nb8)
    i_s16 = diagmask / math.sqrt(2.0)
    diff = ss16 / (nb16 + EPS) - i_s16
    fb8 = jnp.sqrt(_mm(Pair8, jnp.sum(diff * diff, axis=1, keepdims=True)))
    ol_ref[...] = jnp.full((1, 1), 0.0) + jnp.mean(fb8)

    adjz = adj16 * (1.0 - diagmask)
    d16 = jnp.sum(adjz, axis=1, keepdims=True)
    dsq = jnp.sqrt(d16 + EPS)
    dsp = jnp.concatenate([_mm(G0, dsq), _mm(G1, dsq)], axis=1)
    adjn = adjz / (dsq * dsp + EPS)
    adj_ref[...] = adjn

    adjnT = _dd0(adjn, I16)
    cond = (ri16 % 2 == 0)
    sel = jnp.where(cond, jnp.broadcast_to(adjnT[0:1, :], (16, 16)),
                    jnp.broadcast_to(adjnT[1:2, :], (16, 16)))
    M = blockmask * sel
    agg2 = _mm(M, accO)
    h2 = _mm(agg2, wr3_ref[...]) + br3_ref[...] + _mm(accO, wo3_ref[...])
    xg = 0.5 * _mm(Pair8, h2)
    xg = jnp.maximum(_mm(xg, w1_ref[...]) + b1_ref[...], 0.0)
    logits = _mm(xg, w2_ref[...]) + b2_ref[...]
    m = jnp.max(logits, axis=1, keepdims=True)
    lse = m + jnp.log(jnp.sum(jnp.exp(logits - m), axis=1, keepdims=True))
    logp_ref[...] = logits - lse


def _tc_c(accO, accM, W_rel3, b_rel3, W_root3, W1, b1, W2, b2):
    full = lambda shp: pl.BlockSpec(shp, lambda: tuple(0 for _ in shp))
    return pl.pallas_call(
        _tcc_body,
        grid=(),
        in_specs=[
            full((16, H)), full((16, 128)),
            full((H, H)), full((1, H)), full((H, H)),
            full((H, H)), full((1, H)), full((H, OUT)), full((1, OUT)),
        ],
        out_specs=[full((B, OUT)), full((1, 1)), full((1, 1)), full((16, C))],
        out_shape=[
            jax.ShapeDtypeStruct((B, OUT), jnp.float32),
            jax.ShapeDtypeStruct((1, 1), jnp.float32),
            jax.ShapeDtypeStruct((1, 1), jnp.float32),
            jax.ShapeDtypeStruct((16, C), jnp.float32),
        ],
        interpret=_INTERPRET,
    )(accO, accM, W_rel3, b_rel3.reshape(1, H), W_root3,
      W1, b1.reshape(1, H), W2, b2.reshape(1, OUT))


def kernel(x, edge_index, batch, edge_weight, W_rel1, b_rel1, W_root1, Wp, bp,
           W_rel3, b_rel3, W_root3, W1, b1, W2, b2):
    src = edge_index[0]
    dst = edge_index[1]
    msg = x[src] * edge_weight[:, None]
    agg = jax.ops.segment_sum(msg, dst, num_segments=N)[None]
    deg = jax.ops.segment_sum(edge_weight, dst, num_segments=N)
    deg = deg.reshape(1, N, 1)

    h, st, s_logits = _tc_a(agg, x, W_rel1, b_rel1, W_root1, Wp, bp)

    s = st[:, :C]
    q = jax.ops.segment_sum(edge_weight[:, None] * s[dst], src,
                            num_segments=N)[None]

    oh2 = (batch[:, None] == (jnp.arange(16) // 2)).astype(jnp.float32)
    accO, accM = _tc_b(h, st, oh2, q, deg)
    logp, mc, ol, adjn = _tc_c(accO, accM, W_rel3, b_rel3, W_root3,
                               W1, b1, W2, b2)
    return (logp, mc.reshape(()), ol.reshape(()), s_logits,
            adjn.reshape(B, C, C))


# speedup vs baseline: 32.1311x; 3.8276x over previous
import functools
import math

import jax
import jax.numpy as jnp
from jax import lax
from jax.experimental import pallas as pl
from jax.experimental.pallas import tpu as pltpu
from jax.experimental.pallas import tpu_sc as plsc

N = 10000
E = 320000
F_IN = 128
H = 512
C = 2
B = 8
OUT = 10
EPS = 1e-15
BLK = 1000
NB = N // BLK

NW = 32
CH = 48
NCHUNK = 216
NSUP = NCHUNK // 8
SCH = 8 * CH
EPT = CH * NCHUNK
EPAD = NW * EPT
G16 = CH // 16

_INTERPRET = False


def _dd0(a, b):
    return jax.lax.dot_general(a, b, (((0,), (0,)), ((), ())),
                               preferred_element_type=jnp.float32)


def _mm(a, b):
    return jnp.dot(a, b, preferred_element_type=jnp.float32)


def _mk_sc_body(F, with_deg):
    def body(table_hbm, gidxr_hbm, sidxr_hbm, wr_hbm, zrows_hbm, *rest):
        if with_deg:
            (zn_hbm, acc_hbm, deg_hbm, gb, sb, wb, rows_v, gsem, ssem, isem,
             acc_sh, deg_sh) = rest
        else:
            acc_hbm, gb, sb, wb, rows_v, gsem, ssem, isem, acc_sh = rest
        c = lax.axis_index("c")
        s = lax.axis_index("s")
        wid = c * 16 + s

        r0 = pl.multiple_of(s * 640, 8)
        for k5 in range(5):
            off = pl.multiple_of(r0 + k5 * 128, 8)

            @pl.when(off + 128 <= N)
            def _():
                pltpu.sync_copy(zrows_hbm, acc_sh.at[pl.ds(off, 128)])

        @pl.when(s == 15)
        def _():
            pltpu.sync_copy(zrows_hbm.at[pl.ds(0, 16)],
                            acc_sh.at[pl.ds(N - 16, 16)])

        if with_deg:
            @pl.when(s == 0)
            def _():
                pltpu.sync_copy(zn_hbm, deg_sh)

        plsc.subcore_barrier()

        def i_copies(u, p):
            usl = pl.ds(pl.multiple_of(u * SCH, 8), SCH)
            psl = pl.ds(pl.multiple_of(p * SCH, 8), SCH)
            return (
                pltpu.make_async_copy(gidxr_hbm.at[wid, usl], gb.at[psl],
                                      isem.at[p]),
                pltpu.make_async_copy(sidxr_hbm.at[wid, usl], sb.at[psl],
                                      isem.at[p]),
                pltpu.make_async_copy(wr_hbm.at[wid, usl], wb.at[psl],
                                      isem.at[p]),
            )

        def i_issue(u, p):
            for cp in i_copies(u, p):
                cp.start()

        def i_wait(u, p):
            for cp in i_copies(u, p):
                cp.wait()

        def _off(j):
            p, k8 = (j // 8) % 2, j % 8
            return p * SCH + k8 * CH

        def g_copy(j):
            b = j % 4
            return pltpu.make_async_copy(
                table_hbm.at[gb.at[pl.ds(pl.multiple_of(_off(j), 8), CH)]],
                rows_v.at[b], gsem.at[b])

        def s_copies(j):
            b = j % 4
            off = _off(j)
            out = []
            for l in range(CH // 16):
                lsl = pl.ds(pl.multiple_of(off + l * 16, 8), 16)
                d16 = sb[lsl]
                out.append(pltpu.make_async_copy(
                    rows_v.at[b, pl.ds(l * 16, 16)], acc_sh.at[d16],
                    ssem.at[b]))
                if with_deg:
                    out.append(pltpu.make_async_copy(
                        wb.at[lsl], deg_sh.at[d16], ssem.at[b]))
            return out

        def s_issue(j):
            b = j % 4
            off = _off(j)
            for l in range(CH // 16):
                lsl = pl.ds(pl.multiple_of(off + l * 16, 8), 16)
                d16 = sb[lsl]
                pltpu.async_copy(rows_v.at[b, pl.ds(l * 16, 16)],
                                 acc_sh.at[d16], ssem.at[b], add=True)
                if with_deg:
                    pltpu.async_copy(wb.at[lsl], deg_sh.at[d16],
                                     ssem.at[b], add=True)

        def s_wait(j):
            for cp in s_copies(j):
                cp.wait()

        dnums = lax.GatherDimensionNumbers(offset_dims=(),
                                           collapsed_slice_dims=(0,),
                                           start_index_map=(0,))

        def _splat(v16, l):
            idx = jnp.full((16, 1), l, jnp.int32)
            return lax.gather(v16, idx, dnums, (1,),
                              mode=lax.GatherScatterMode.PROMISE_IN_BOUNDS)

        def scale(j):
            b, p, k8 = j % 4, (j // 8) % 2, j % 8
            rb = rows_v.at[b]

            def gbody(g, carry):
                wv = wb[pl.ds(pl.multiple_of(p * SCH + k8 * CH + g * 16, 8),
                              16)]
                for l in range(16):
                    e = g * 16 + l
                    ws = _splat(wv, l)
                    for k in range(F // 16):
                        sl_ = pl.ds(k * 16, 16)
                        rb[e, sl_] = rb[e, sl_] * ws
                return carry

            lax.fori_loop(0, G16, gbody, 0)

        def chunk(j, *, peeled):
            u = j // 8

            if not peeled:
                @pl.when((j % 8 == 5) & (j < 8 * (NSUP - 1)))
                def _():
                    i_wait(u + 1, (u + 1) % 2)

            g_copy(j).wait()
            scale(j)
            s_issue(j)
            if not peeled:
                s_wait(j - 2)

                @pl.when(j + 2 <= NCHUNK - 1)
                def _():
                    g_copy(j + 2).start()

                @pl.when((j % 8 == 1) & (j >= 9) & (u <= NSUP - 2))
                def _():
                    i_issue(u + 1, (u + 1) % 2)
            else:
                g_copy(j + 2).start()

        i_issue(0, 0)
        i_wait(0, 0)
        i_issue(1, 1)
        g_copy(0).start()
        g_copy(1).start()
        chunk(0, peeled=True)
        chunk(1, peeled=True)

        def jbody(j, carry):
            chunk(j, peeled=False)
            return carry

        lax.fori_loop(2, NCHUNK, jbody, 0)

        s_wait(NCHUNK - 2)
        s_wait(NCHUNK - 1)

        plsc.subcore_barrier()
        for k5 in range(5):
            off = pl.multiple_of(r0 + k5 * 128, 8)

            @pl.when(off + 128 <= N)
            def _():
                pltpu.sync_copy(acc_sh.at[pl.ds(off, 128)],
                                acc_hbm.at[c, pl.ds(off, 128)])

        @pl.when(s == 15)
        def _():
            pltpu.sync_copy(acc_sh.at[pl.ds(N - 16, 16)],
                            acc_hbm.at[c, pl.ds(N - 16, 16)])

        if with_deg:
            @pl.when(s == 0)
            def _():
                pltpu.sync_copy(deg_sh, deg_hbm.at[c])

    return body


def _sc_pass(table, gidxr, sidxr, wr, F, with_deg):
    mesh = plsc.VectorSubcoreMesh(core_axis_name="c", subcore_axis_name="s")
    out_type = [jax.ShapeDtypeStruct((2, N, F), jnp.float32)]
    if with_deg:
        out_type.append(jax.ShapeDtypeStruct((2, N), jnp.float32))
    scratch = [
        pltpu.VMEM((2 * SCH,), jnp.int32),
        pltpu.VMEM((2 * SCH,), jnp.int32),
        pltpu.VMEM((2 * SCH,), jnp.float32),
        pltpu.VMEM((4, CH, F), jnp.float32),
        pltpu.SemaphoreType.DMA((4,)),
        pltpu.SemaphoreType.DMA((4,)),
        pltpu.SemaphoreType.DMA((2,)),
        pltpu.VMEM_SHARED((N, F), jnp.float32),
    ]
    if with_deg:
        scratch.append(pltpu.VMEM_SHARED((N,), jnp.float32))
    f = pl.kernel(
        _mk_sc_body(F, with_deg),
        out_type=out_type,
        mesh=mesh,
        scratch_types=scratch,
        compiler_params=pltpu.CompilerParams(
            use_tc_tiling_on_sc=(F == F_IN)),
    )
    args = [table, gidxr, sidxr, wr, jnp.zeros((128, F), jnp.float32)]
    if with_deg:
        args.append(jnp.zeros((N,), jnp.float32))
    out = f(*args)
    return out if with_deg else out[0]


def _tca_body(agg_ref, x_ref, wrel_ref, brel_ref, wroot_ref, wp_ref, bp_ref,
              h_ref, st_ref, sl_ref):
    agg = jnp.sum(agg_ref[...], axis=0)
    h = _mm(agg, wrel_ref[...]) + _mm(x_ref[...], wroot_ref[...]) + brel_ref[...]
    h = jnp.maximum(h, 0.0)
    sl = _mm(h, wp_ref[...]) + bp_ref[...]
    m = jnp.max(sl, axis=1, keepdims=True)
    e = jnp.exp(sl - m)
    s = e / jnp.sum(e, axis=1, keepdims=True)
    h_ref[...] = h
    sl_ref[...] = sl
    st_ref[...] = jnp.concatenate([s] * 8, axis=1)


def _tc_a(agg_parts, x, W_rel1, b_rel1, W_root1, Wp, bp):
    K = agg_parts.shape[0]
    return pl.pallas_call(
        _tca_body,
        grid=(NB,),
        in_specs=[
            pl.BlockSpec((K, BLK, F_IN), lambda i: (0, i, 0)),
            pl.BlockSpec((BLK, F_IN), lambda i: (i, 0)),
            pl.BlockSpec((F_IN, H), lambda i: (0, 0)),
            pl.BlockSpec((1, H), lambda i: (0, 0)),
            pl.BlockSpec((F_IN, H), lambda i: (0, 0)),
            pl.BlockSpec((H, C), lambda i: (0, 0)),
            pl.BlockSpec((1, C), lambda i: (0, 0)),
        ],
        out_specs=[
            pl.BlockSpec((BLK, H), lambda i: (i, 0)),
            pl.BlockSpec((BLK, 16), lambda i: (i, 0)),
            pl.BlockSpec((BLK, C), lambda i: (i, 0)),
        ],
        out_shape=[
            jax.ShapeDtypeStruct((N, H), jnp.float32),
            jax.ShapeDtypeStruct((N, 16), jnp.float32),
            jax.ShapeDtypeStruct((N, C), jnp.float32),
        ],
        interpret=_INTERPRET,
    )(agg_parts, x, W_rel1, b_rel1.reshape(1, H), W_root1, Wp, bp.reshape(1, C))


def _tcb_body(h_ref, st_ref, oh2_ref, q_ref, deg_ref, accO_ref, accM_ref):
    i = pl.program_id(0)
    P = oh2_ref[...] * st_ref[...]
    s = st_ref[...][:, 0:C]
    q = jnp.sum(q_ref[...], axis=0)[:, 0:C]
    deg = jnp.sum(deg_ref[...], axis=0)
    o16 = _dd0(P, h_ref[...])
    ss16 = _dd0(P, s)
    adj16 = _dd0(P, q)
    degssq = deg * jnp.sum(s * s, axis=1, keepdims=True)
    den16 = _dd0(oh2_ref[...], degssq)

    @pl.when(i == 0)
    def _():
        accO_ref[...] = jnp.zeros_like(accO_ref)
        accM_ref[...] = jnp.zeros_like(accM_ref)

    accO_ref[...] += o16
    accM_ref[:, 0:2] += ss16
    accM_ref[:, 2:4] += adj16
    accM_ref[:, 4:5] += den16


def _tc_b(h, st, oh2, q_parts, deg_parts):
    Kq = q_parts.shape[0]
    Kd = deg_parts.shape[0]
    return pl.pallas_call(
        _tcb_body,
        grid=(NB,),
        in_specs=[
            pl.BlockSpec((BLK, H), lambda i: (i, 0)),
            pl.BlockSpec((BLK, 16), lambda i: (i, 0)),
            pl.BlockSpec((BLK, 16), lambda i: (i, 0)),
            pl.BlockSpec((Kq, BLK, 16), lambda i: (0, i, 0)),
            pl.BlockSpec((Kd, BLK, 1), lambda i: (0, i, 0)),
        ],
        out_specs=[
            pl.BlockSpec((16, H), lambda i: (0, 0)),
            pl.BlockSpec((16, 128), lambda i: (0, 0)),
        ],
        out_shape=[
            jax.ShapeDtypeStruct((16, H), jnp.float32),
            jax.ShapeDtypeStruct((16, 128), jnp.float32),
        ],
        interpret=_INTERPRET,
    )(h, st, oh2, q_parts, deg_parts)


def _tcc_body(accO_ref, accM_ref, wr3_ref, br3_ref, wo3_ref, w1_ref, b1_ref,
              w2_ref, b2_ref, logp_ref, mc_ref, ol_ref, adj_ref):
    accO = accO_ref[...]
    accM = accM_ref[...]
    ss16 = accM[:, 0:2]
    adj16 = accM[:, 2:4]
    den16 = accM[:, 4:5]

    ri16 = jax.lax.broadcasted_iota(jnp.int32, (16, 16), 0)
    ci16 = jax.lax.broadcasted_iota(jnp.int32, (16, 16), 1)
    I16 = (ri16 == ci16).astype(jnp.float32)
    blockmask = (ri16 // 2 == ci16 // 2).astype(jnp.float32)
    G0 = (ci16 == 2 * (ri16 // 2)).astype(jnp.float32)
    G1 = (ci16 == 2 * (ri16 // 2) + 1).astype(jnp.float32)
    ri82 = jax.lax.broadcasted_iota(jnp.int32, (8, 16), 0)
    ci82 = jax.lax.broadcasted_iota(jnp.int32, (8, 16), 1)
    Pair8 = (ci82 // 2 == ri82).astype(jnp.float32)
    Geven = (ci82 == 2 * ri82).astype(jnp.float32)
    ri168 = jax.lax.broadcasted_iota(jnp.int32, (16, 8), 0)
    ci168 = jax.lax.broadcasted_iota(jnp.int32, (16, 8), 1)
    G2 = (ri168 // 2 == ci168).astype(jnp.float32)
    ri2 = jax.lax.broadcasted_iota(jnp.int32, (16, 2), 0)
    ci2 = jax.lax.broadcasted_iota(jnp.int32, (16, 2), 1)
    diagmask = (ci2 == ri2 % 2).astype(jnp.float32)

    diag16 = jnp.sum(adj16 * diagmask, axis=1, keepdims=True)
    num8 = _mm(Pair8, diag16)
    den8 = _mm(Geven, den16)
    mc = -jnp.mean(num8 / (den8 + EPS))
    mc_ref[...] = jnp.full((1, 1), 0.0) + mc

    rs = jnp.sum(ss16 * ss16, axis=1, keepdims=True)
    nb8 = jnp.sqrt(_mm(Pair8, rs))
    nb16 = _mm(G2, nb8)
    i_s16 = diagmask / math.sqrt(2.0)
    diff = ss16 / (nb16 + EPS) - i_s16
    fb8 = jnp.sqrt(_mm(Pair8, jnp.sum(diff * diff, axis=1, keepdims=True)))
    ol_ref[...] = jnp.full((1, 1), 0.0) + jnp.mean(fb8)

    adjz = adj16 * (1.0 - diagmask)
    d16 = jnp.sum(adjz, axis=1, keepdims=True)
    dsq = jnp.sqrt(d16 + EPS)
    dsp = jnp.concatenate([_mm(G0, dsq), _mm(G1, dsq)], axis=1)
    adjn = adjz / (dsq * dsp + EPS)
    adj_ref[...] = adjn

    adjnT = _dd0(adjn, I16)
    cond = (ri16 % 2 == 0)
    sel = jnp.where(cond, jnp.broadcast_to(adjnT[0:1, :], (16, 16)),
                    jnp.broadcast_to(adjnT[1:2, :], (16, 16)))
    M = blockmask * sel
    agg2 = _mm(M, accO)
    h2 = _mm(agg2, wr3_ref[...]) + br3_ref[...] + _mm(accO, wo3_ref[...])
    xg = 0.5 * _mm(Pair8, h2)
    xg = jnp.maximum(_mm(xg, w1_ref[...]) + b1_ref[...], 0.0)
    logits = _mm(xg, w2_ref[...]) + b2_ref[...]
    m = jnp.max(logits, axis=1, keepdims=True)
    lse = m + jnp.log(jnp.sum(jnp.exp(logits - m), axis=1, keepdims=True))
    logp_ref[...] = logits - lse


def _tc_c(accO, accM, W_rel3, b_rel3, W_root3, W1, b1, W2, b2):
    full = lambda shp: pl.BlockSpec(shp, lambda: tuple(0 for _ in shp))
    return pl.pallas_call(
        _tcc_body,
        grid=(),
        in_specs=[
            full((16, H)), full((16, 128)),
            full((H, H)), full((1, H)), full((H, H)),
            full((H, H)), full((1, H)), full((H, OUT)), full((1, OUT)),
        ],
        out_specs=[full((B, OUT)), full((1, 1)), full((1, 1)), full((16, C))],
        out_shape=[
            jax.ShapeDtypeStruct((B, OUT), jnp.float32),
            jax.ShapeDtypeStruct((1, 1), jnp.float32),
            jax.ShapeDtypeStruct((1, 1), jnp.float32),
            jax.ShapeDtypeStruct((16, C), jnp.float32),
        ],
        interpret=_INTERPRET,
    )(accO, accM, W_rel3, b_rel3.reshape(1, H), W_root3,
      W1, b1.reshape(1, H), W2, b2.reshape(1, OUT))


def kernel(x, edge_index, batch, edge_weight, W_rel1, b_rel1, W_root1, Wp, bp,
           W_rel3, b_rel3, W_root3, W1, b1, W2, b2):
    src = edge_index[0]
    dst = edge_index[1]
    pad = EPAD - E
    srcr = jnp.concatenate([src, jnp.zeros((pad,), src.dtype)]).reshape(NW, EPT)
    dstr = jnp.concatenate([dst, jnp.zeros((pad,), dst.dtype)]).reshape(NW, EPT)
    wr = jnp.concatenate([edge_weight,
                          jnp.zeros((pad,), jnp.float32)]).reshape(NW, EPT)
    agg, deg2 = _sc_pass(x, srcr, dstr, wr, F_IN, True)
    deg = deg2.reshape(2, N, 1)

    h, st, s_logits = _tc_a(agg, x, W_rel1, b_rel1, W_root1, Wp, bp)

    q16 = _sc_pass(st, dstr, srcr, wr, 16, False)

    oh2 = (batch[:, None] == (jnp.arange(16) // 2)).astype(jnp.float32)
    accO, accM = _tc_b(h, st, oh2, q16, deg)
    logp, mc, ol, adjn = _tc_c(accO, accM, W_rel3, b_rel3, W_root3,
                               W1, b1, W2, b2)
    return (logp, mc.reshape(()), ol.reshape(()), s_logits,
            adjn.reshape(B, C, C))


# speedup vs baseline: 32.3344x; 1.0063x over previous
import functools
import math

import jax
import jax.numpy as jnp
from jax import lax
from jax.experimental import pallas as pl
from jax.experimental.pallas import tpu as pltpu
from jax.experimental.pallas import tpu_sc as plsc

N = 10000
E = 320000
F_IN = 128
H = 512
C = 2
B = 8
OUT = 10
EPS = 1e-15
BLK = 1000
NB = N // BLK

NW = 32
CH = 48
NCHUNK = 216
NSUP = NCHUNK // 8
SCH = 8 * CH
EPT = CH * NCHUNK
EPAD = NW * EPT
G16 = CH // 16

_INTERPRET = False


def _dd0(a, b):
    return jax.lax.dot_general(a, b, (((0,), (0,)), ((), ())),
                               preferred_element_type=jnp.float32)


def _mm(a, b):
    return jnp.dot(a, b, preferred_element_type=jnp.float32)


def _mk_sc_body(F, with_deg):
    def body(table_hbm, gidxr_hbm, sidxr_hbm, wr_hbm, zrows_hbm, *rest):
        if with_deg:
            (zn_hbm, acc_hbm, deg_hbm, gb, sb, wb, rows_v, gsem, ssem, isem,
             acc_sh, deg_sh) = rest
        else:
            acc_hbm, gb, sb, wb, rows_v, gsem, ssem, isem, acc_sh = rest
        c = lax.axis_index("c")
        s = lax.axis_index("s")
        wid = c * 16 + s

        r0 = pl.multiple_of(s * 640, 8)
        for k5 in range(5):
            off = pl.multiple_of(r0 + k5 * 128, 8)

            @pl.when(off + 128 <= N)
            def _():
                pltpu.sync_copy(zrows_hbm, acc_sh.at[pl.ds(off, 128)])

        @pl.when(s == 15)
        def _():
            pltpu.sync_copy(zrows_hbm.at[pl.ds(0, 16)],
                            acc_sh.at[pl.ds(N - 16, 16)])

        if with_deg:
            @pl.when(s == 0)
            def _():
                pltpu.sync_copy(zn_hbm, deg_sh)

        plsc.subcore_barrier()

        def i_copies(u, p):
            usl = pl.ds(pl.multiple_of(u * SCH, 8), SCH)
            psl = pl.ds(pl.multiple_of(p * SCH, 8), SCH)
            return (
                pltpu.make_async_copy(gidxr_hbm.at[wid, usl], gb.at[psl],
                                      isem.at[p]),
                pltpu.make_async_copy(sidxr_hbm.at[wid, u],
                                      sb.at[pl.ds(p * 8, 8)], isem.at[p]),
                pltpu.make_async_copy(wr_hbm.at[wid, usl], wb.at[psl],
                                      isem.at[p]),
            )

        def i_issue(u, p):
            for cp in i_copies(u, p):
                cp.start()

        def i_wait(u, p):
            for cp in i_copies(u, p):
                cp.wait()

        def _off(j):
            p, k8 = (j // 8) % 2, j % 8
            return p * SCH + k8 * CH

        def g_copy(j):
            b = j % 4
            return pltpu.make_async_copy(
                table_hbm.at[gb.at[pl.ds(pl.multiple_of(_off(j), 8), CH)]],
                rows_v.at[b], gsem.at[b])

        def _srow(j):
            p, k8 = (j // 8) % 2, j % 8
            return sb.at[p * 8 + k8]

        def s_issue(j):
            b = j % 4
            pltpu.async_copy(rows_v.at[b], acc_sh.at[_srow(j)], ssem.at[b],
                             add=True)
            if with_deg:
                pltpu.async_copy(
                    wb.at[pl.ds(pl.multiple_of(_off(j), 8), CH)],
                    deg_sh.at[_srow(j)], ssem.at[b], add=True)

        def s_wait(j):
            b = j % 4
            pltpu.make_async_copy(rows_v.at[b], acc_sh.at[_srow(j)],
                                  ssem.at[b]).wait()
            if with_deg:
                pltpu.make_async_copy(
                    wb.at[pl.ds(pl.multiple_of(_off(j), 8), CH)],
                    deg_sh.at[_srow(j)], ssem.at[b]).wait()

        dnums = lax.GatherDimensionNumbers(offset_dims=(),
                                           collapsed_slice_dims=(0,),
                                           start_index_map=(0,))

        def _splat(v16, l):
            idx = jnp.full((16, 1), l, jnp.int32)
            return lax.gather(v16, idx, dnums, (1,),
                              mode=lax.GatherScatterMode.PROMISE_IN_BOUNDS)

        def scale(j):
            b, p, k8 = j % 4, (j // 8) % 2, j % 8
            rb = rows_v.at[b]

            def gbody(g, carry):
                wv = wb[pl.ds(pl.multiple_of(p * SCH + k8 * CH + g * 16, 8),
                              16)]
                for l in range(16):
                    e = g * 16 + l
                    ws = _splat(wv, l)
                    for k in range(F // 16):
                        sl_ = pl.ds(k * 16, 16)
                        rb[e, sl_] = rb[e, sl_] * ws
                return carry

            lax.fori_loop(0, G16, gbody, 0)

        def chunk(j, *, peeled):
            u = j // 8

            if not peeled:
                @pl.when((j % 8 == 5) & (j < 8 * (NSUP - 1)))
                def _():
                    i_wait(u + 1, (u + 1) % 2)

            g_copy(j).wait()
            scale(j)
            s_issue(j)
            if not peeled:
                s_wait(j - 2)

                @pl.when(j + 2 <= NCHUNK - 1)
                def _():
                    g_copy(j + 2).start()

                @pl.when((j % 8 == 1) & (j >= 9) & (u <= NSUP - 2))
                def _():
                    i_issue(u + 1, (u + 1) % 2)
            else:
                g_copy(j + 2).start()

        i_issue(0, 0)
        i_wait(0, 0)
        i_issue(1, 1)
        g_copy(0).start()
        g_copy(1).start()
        chunk(0, peeled=True)
        chunk(1, peeled=True)

        def jbody(j, carry):
            chunk(j, peeled=False)
            return carry

        lax.fori_loop(2, NCHUNK, jbody, 0)

        s_wait(NCHUNK - 2)
        s_wait(NCHUNK - 1)

        plsc.subcore_barrier()
        for k5 in range(5):
            off = pl.multiple_of(r0 + k5 * 128, 8)

            @pl.when(off + 128 <= N)
            def _():
                pltpu.sync_copy(acc_sh.at[pl.ds(off, 128)],
                                acc_hbm.at[c, pl.ds(off, 128)])

        @pl.when(s == 15)
        def _():
            pltpu.sync_copy(acc_sh.at[pl.ds(N - 16, 16)],
                            acc_hbm.at[c, pl.ds(N - 16, 16)])

        if with_deg:
            @pl.when(s == 0)
            def _():
                pltpu.sync_copy(deg_sh, deg_hbm.at[c])

    return body


def _sc_pass(table, gidxr, sidxr, wr, F, with_deg):
    mesh = plsc.VectorSubcoreMesh(core_axis_name="c", subcore_axis_name="s")
    out_type = [jax.ShapeDtypeStruct((2, N, F), jnp.float32)]
    if with_deg:
        out_type.append(jax.ShapeDtypeStruct((2, N), jnp.float32))
    scratch = [
        pltpu.VMEM((2 * SCH,), jnp.int32),
        pltpu.VMEM((16, CH), jnp.int32),
        pltpu.VMEM((2 * SCH,), jnp.float32),
        pltpu.VMEM((4, CH, F), jnp.float32),
        pltpu.SemaphoreType.DMA((4,)),
        pltpu.SemaphoreType.DMA((4,)),
        pltpu.SemaphoreType.DMA((2,)),
        pltpu.VMEM_SHARED((N, F), jnp.float32),
    ]
    if with_deg:
        scratch.append(pltpu.VMEM_SHARED((N,), jnp.float32))
    f = pl.kernel(
        _mk_sc_body(F, with_deg),
        out_type=out_type,
        mesh=mesh,
        scratch_types=scratch,
        compiler_params=pltpu.CompilerParams(use_tc_tiling_on_sc=False),
    )
    args = [table, gidxr, sidxr, wr, jnp.zeros((128, F), jnp.float32)]
    if with_deg:
        args.append(jnp.zeros((N,), jnp.float32))
    out = f(*args)
    return out if with_deg else out[0]


def _tca_body(agg_ref, x_ref, wrel_ref, brel_ref, wroot_ref, wp_ref, bp_ref,
              h_ref, st_ref, sl_ref):
    agg = jnp.sum(agg_ref[...], axis=0)
    h = _mm(agg, wrel_ref[...]) + _mm(x_ref[...], wroot_ref[...]) + brel_ref[...]
    h = jnp.maximum(h, 0.0)
    sl = _mm(h, wp_ref[...]) + bp_ref[...]
    m = jnp.max(sl, axis=1, keepdims=True)
    e = jnp.exp(sl - m)
    s = e / jnp.sum(e, axis=1, keepdims=True)
    h_ref[...] = h
    sl_ref[...] = sl
    st_ref[...] = jnp.concatenate([s] * 8, axis=1)


def _tc_a(agg_parts, x, W_rel1, b_rel1, W_root1, Wp, bp):
    K = agg_parts.shape[0]
    return pl.pallas_call(
        _tca_body,
        grid=(NB,),
        in_specs=[
            pl.BlockSpec((K, BLK, F_IN), lambda i: (0, i, 0)),
            pl.BlockSpec((BLK, F_IN), lambda i: (i, 0)),
            pl.BlockSpec((F_IN, H), lambda i: (0, 0)),
            pl.BlockSpec((1, H), lambda i: (0, 0)),
            pl.BlockSpec((F_IN, H), lambda i: (0, 0)),
            pl.BlockSpec((H, C), lambda i: (0, 0)),
            pl.BlockSpec((1, C), lambda i: (0, 0)),
        ],
        out_specs=[
            pl.BlockSpec((BLK, H), lambda i: (i, 0)),
            pl.BlockSpec((BLK, 16), lambda i: (i, 0)),
            pl.BlockSpec((BLK, C), lambda i: (i, 0)),
        ],
        out_shape=[
            jax.ShapeDtypeStruct((N, H), jnp.float32),
            jax.ShapeDtypeStruct((N, 16), jnp.float32),
            jax.ShapeDtypeStruct((N, C), jnp.float32),
        ],
        interpret=_INTERPRET,
    )(agg_parts, x, W_rel1, b_rel1.reshape(1, H), W_root1, Wp, bp.reshape(1, C))


def _tcb_body(h_ref, st_ref, oh2_ref, q_ref, deg_ref, accO_ref, accM_ref):
    i = pl.program_id(0)
    P = oh2_ref[...] * st_ref[...]
    s = st_ref[...][:, 0:C]
    q = jnp.sum(q_ref[...], axis=0)[:, 0:C]
    deg = jnp.sum(deg_ref[...], axis=0)
    o16 = _dd0(P, h_ref[...])
    ss16 = _dd0(P, s)
    adj16 = _dd0(P, q)
    degssq = deg * jnp.sum(s * s, axis=1, keepdims=True)
    den16 = _dd0(oh2_ref[...], degssq)

    @pl.when(i == 0)
    def _():
        accO_ref[...] = jnp.zeros_like(accO_ref)
        accM_ref[...] = jnp.zeros_like(accM_ref)

    accO_ref[...] += o16
    accM_ref[:, 0:2] += ss16
    accM_ref[:, 2:4] += adj16
    accM_ref[:, 4:5] += den16


def _tc_b(h, st, oh2, q_parts, deg_parts):
    Kq = q_parts.shape[0]
    Kd = deg_parts.shape[0]
    return pl.pallas_call(
        _tcb_body,
        grid=(NB,),
        in_specs=[
            pl.BlockSpec((BLK, H), lambda i: (i, 0)),
            pl.BlockSpec((BLK, 16), lambda i: (i, 0)),
            pl.BlockSpec((BLK, 16), lambda i: (i, 0)),
            pl.BlockSpec((Kq, BLK, 16), lambda i: (0, i, 0)),
            pl.BlockSpec((Kd, BLK, 1), lambda i: (0, i, 0)),
        ],
        out_specs=[
            pl.BlockSpec((16, H), lambda i: (0, 0)),
            pl.BlockSpec((16, 128), lambda i: (0, 0)),
        ],
        out_shape=[
            jax.ShapeDtypeStruct((16, H), jnp.float32),
            jax.ShapeDtypeStruct((16, 128), jnp.float32),
        ],
        interpret=_INTERPRET,
    )(h, st, oh2, q_parts, deg_parts)


def _tcc_body(accO_ref, accM_ref, wr3_ref, br3_ref, wo3_ref, w1_ref, b1_ref,
              w2_ref, b2_ref, logp_ref, mc_ref, ol_ref, adj_ref):
    accO = accO_ref[...]
    accM = accM_ref[...]
    ss16 = accM[:, 0:2]
    adj16 = accM[:, 2:4]
    den16 = accM[:, 4:5]

    ri16 = jax.lax.broadcasted_iota(jnp.int32, (16, 16), 0)
    ci16 = jax.lax.broadcasted_iota(jnp.int32, (16, 16), 1)
    I16 = (ri16 == ci16).astype(jnp.float32)
    blockmask = (ri16 // 2 == ci16 // 2).astype(jnp.float32)
    G0 = (ci16 == 2 * (ri16 // 2)).astype(jnp.float32)
    G1 = (ci16 == 2 * (ri16 // 2) + 1).astype(jnp.float32)
    ri82 = jax.lax.broadcasted_iota(jnp.int32, (8, 16), 0)
    ci82 = jax.lax.broadcasted_iota(jnp.int32, (8, 16), 1)
    Pair8 = (ci82 // 2 == ri82).astype(jnp.float32)
    Geven = (ci82 == 2 * ri82).astype(jnp.float32)
    ri168 = jax.lax.broadcasted_iota(jnp.int32, (16, 8), 0)
    ci168 = jax.lax.broadcasted_iota(jnp.int32, (16, 8), 1)
    G2 = (ri168 // 2 == ci168).astype(jnp.float32)
    ri2 = jax.lax.broadcasted_iota(jnp.int32, (16, 2), 0)
    ci2 = jax.lax.broadcasted_iota(jnp.int32, (16, 2), 1)
    diagmask = (ci2 == ri2 % 2).astype(jnp.float32)

    diag16 = jnp.sum(adj16 * diagmask, axis=1, keepdims=True)
    num8 = _mm(Pair8, diag16)
    den8 = _mm(Geven, den16)
    mc = -jnp.mean(num8 / (den8 + EPS))
    mc_ref[...] = jnp.full((1, 1), 0.0) + mc

    rs = jnp.sum(ss16 * ss16, axis=1, keepdims=True)
    nb8 = jnp.sqrt(_mm(Pair8, rs))
    nb16 = _mm(G2, nb8)
    i_s16 = diagmask / math.sqrt(2.0)
    diff = ss16 / (nb16 + EPS) - i_s16
    fb8 = jnp.sqrt(_mm(Pair8, jnp.sum(diff * diff, axis=1, keepdims=True)))
    ol_ref[...] = jnp.full((1, 1), 0.0) + jnp.mean(fb8)

    adjz = adj16 * (1.0 - diagmask)
    d16 = jnp.sum(adjz, axis=1, keepdims=True)
    dsq = jnp.sqrt(d16 + EPS)
    dsp = jnp.concatenate([_mm(G0, dsq), _mm(G1, dsq)], axis=1)
    adjn = adjz / (dsq * dsp + EPS)
    adj_ref[...] = adjn

    adjnT = _dd0(adjn, I16)
    cond = (ri16 % 2 == 0)
    sel = jnp.where(cond, jnp.broadcast_to(adjnT[0:1, :], (16, 16)),
                    jnp.broadcast_to(adjnT[1:2, :], (16, 16)))
    M = blockmask * sel
    agg2 = _mm(M, accO)
    h2 = _mm(agg2, wr3_ref[...]) + br3_ref[...] + _mm(accO, wo3_ref[...])
    xg = 0.5 * _mm(Pair8, h2)
    xg = jnp.maximum(_mm(xg, w1_ref[...]) + b1_ref[...], 0.0)
    logits = _mm(xg, w2_ref[...]) + b2_ref[...]
    m = jnp.max(logits, axis=1, keepdims=True)
    lse = m + jnp.log(jnp.sum(jnp.exp(logits - m), axis=1, keepdims=True))
    logp_ref[...] = logits - lse


def _tc_c(accO, accM, W_rel3, b_rel3, W_root3, W1, b1, W2, b2):
    full = lambda shp: pl.BlockSpec(shp, lambda: tuple(0 for _ in shp))
    return pl.pallas_call(
        _tcc_body,
        grid=(),
        in_specs=[
            full((16, H)), full((16, 128)),
            full((H, H)), full((1, H)), full((H, H)),
            full((H, H)), full((1, H)), full((H, OUT)), full((1, OUT)),
        ],
        out_specs=[full((B, OUT)), full((1, 1)), full((1, 1)), full((16, C))],
        out_shape=[
            jax.ShapeDtypeStruct((B, OUT), jnp.float32),
            jax.ShapeDtypeStruct((1, 1), jnp.float32),
            jax.ShapeDtypeStruct((1, 1), jnp.float32),
            jax.ShapeDtypeStruct((16, C), jnp.float32),
        ],
        interpret=_INTERPRET,
    )(accO, accM, W_rel3, b_rel3.reshape(1, H), W_root3,
      W1, b1.reshape(1, H), W2, b2.reshape(1, OUT))


def kernel(x, edge_index, batch, edge_weight, W_rel1, b_rel1, W_root1, Wp, bp,
           W_rel3, b_rel3, W_root3, W1, b1, W2, b2):
    src = edge_index[0]
    dst = edge_index[1]
    pad = EPAD - E
    srcp = jnp.concatenate([src, jnp.zeros((pad,), src.dtype)])
    dstp = jnp.concatenate([dst, jnp.zeros((pad,), dst.dtype)])
    wr = jnp.concatenate([edge_weight,
                          jnp.zeros((pad,), jnp.float32)]).reshape(NW, EPT)
    agg, deg2 = _sc_pass(x, srcp.reshape(NW, EPT),
                         dstp.reshape(NW, NSUP, 8, CH), wr, F_IN, True)
    deg = deg2.reshape(2, N, 1)

    h, st, s_logits = _tc_a(agg, x, W_rel1, b_rel1, W_root1, Wp, bp)

    q16 = _sc_pass(st, dstp.reshape(NW, EPT),
                   srcp.reshape(NW, NSUP, 8, CH), wr, 16, False)

    oh2 = (batch[:, None] == (jnp.arange(16) // 2)).astype(jnp.float32)
    accO, accM = _tc_b(h, st, oh2, q16, deg)
    logp, mc, ol, adjn = _tc_c(accO, accM, W_rel3, b_rel3, W_root3,
                               W1, b1, W2, b2)
    return (logp, mc.reshape(()), ol.reshape(()), s_logits,
            adjn.reshape(B, C, C))


# speedup vs baseline: 32.6550x; 1.0099x over previous
import functools
import math

import jax
import jax.numpy as jnp
from jax import lax
from jax.experimental import pallas as pl
from jax.experimental.pallas import tpu as pltpu
from jax.experimental.pallas import tpu_sc as plsc

N = 10000
E = 320000
F_IN = 128
H = 512
C = 2
B = 8
OUT = 10
EPS = 1e-15
BLK = 1000
NB = N // BLK

NW = 32
CH = 48
NCHUNK = 216
NSUP = NCHUNK // 8
SCH = 8 * CH
EPT = CH * NCHUNK
EPAD = NW * EPT
G16 = CH // 16

_INTERPRET = False


def _dd0(a, b):
    return jax.lax.dot_general(a, b, (((0,), (0,)), ((), ())),
                               preferred_element_type=jnp.float32)


def _mm(a, b):
    return jnp.dot(a, b, preferred_element_type=jnp.float32)


def _mk_sc_body(F, with_deg):
    def body(table_hbm, gidxr_hbm, sidxr_hbm, wr_hbm, zrows_hbm, *rest):
        if with_deg:
            (zn_hbm, acc_hbm, deg_hbm, gb, sb, wb, rows_v, gsem, ssem, isem,
             acc_sh, deg_sh) = rest
        else:
            acc_hbm, gb, sb, wb, rows_v, gsem, ssem, isem, acc_sh = rest
        c = lax.axis_index("c")
        s = lax.axis_index("s")
        wid = c * 16 + s

        r0 = pl.multiple_of(s * 640, 8)
        for k5 in range(5):
            off = pl.multiple_of(r0 + k5 * 128, 8)

            @pl.when(off + 128 <= N)
            def _():
                pltpu.sync_copy(zrows_hbm, acc_sh.at[pl.ds(off, 128)])

        @pl.when(s == 15)
        def _():
            pltpu.sync_copy(zrows_hbm.at[pl.ds(0, 16)],
                            acc_sh.at[pl.ds(N - 16, 16)])

        if with_deg:
            @pl.when(s == 0)
            def _():
                pltpu.sync_copy(zn_hbm, deg_sh)

        plsc.subcore_barrier()

        def i_copies(u, p):
            usl = pl.ds(pl.multiple_of(u * SCH, 8), SCH)
            psl = pl.ds(pl.multiple_of(p * SCH, 8), SCH)
            return (
                pltpu.make_async_copy(gidxr_hbm.at[wid, usl], gb.at[psl],
                                      isem.at[p]),
                pltpu.make_async_copy(sidxr_hbm.at[wid, u],
                                      sb.at[pl.ds(p * 8, 8)], isem.at[p]),
                pltpu.make_async_copy(wr_hbm.at[wid, usl], wb.at[psl],
                                      isem.at[p]),
            )

        def i_issue(u, p):
            for cp in i_copies(u, p):
                cp.start()

        def i_wait(u, p):
            for cp in i_copies(u, p):
                cp.wait()

        def _off(j):
            p, k8 = (j // 8) % 2, j % 8
            return p * SCH + k8 * CH

        def g_copy(j):
            b = j % 4
            return pltpu.make_async_copy(
                table_hbm.at[gb.at[pl.ds(pl.multiple_of(_off(j), 8), CH)]],
                rows_v.at[b], gsem.at[b])

        def _srow(j):
            p, k8 = (j // 8) % 2, j % 8
            return sb.at[p * 8 + k8]

        def s_issue(j):
            b = j % 4
            pltpu.async_copy(rows_v.at[b], acc_sh.at[_srow(j)], ssem.at[b],
                             add=True)
            if with_deg:
                pltpu.async_copy(
                    wb.at[pl.ds(pl.multiple_of(_off(j), 8), CH)],
                    deg_sh.at[_srow(j)], ssem.at[b], add=True)

        def s_wait(j):
            b = j % 4
            pltpu.make_async_copy(rows_v.at[b], acc_sh.at[_srow(j)],
                                  ssem.at[b]).wait()
            if with_deg:
                pltpu.make_async_copy(
                    wb.at[pl.ds(pl.multiple_of(_off(j), 8), CH)],
                    deg_sh.at[_srow(j)], ssem.at[b]).wait()

        dnums = lax.GatherDimensionNumbers(offset_dims=(),
                                           collapsed_slice_dims=(0,),
                                           start_index_map=(0,))

        def _splat(v16, l):
            idx = jnp.full((16, 1), l, jnp.int32)
            return lax.gather(v16, idx, dnums, (1,),
                              mode=lax.GatherScatterMode.PROMISE_IN_BOUNDS)

        def scale(j):
            b, p, k8 = j % 4, (j // 8) % 2, j % 8
            rb = rows_v.at[b]

            def gbody(g, carry):
                wv = wb[pl.ds(pl.multiple_of(p * SCH + k8 * CH + g * 16, 8),
                              16)]
                for l in range(16):
                    e = g * 16 + l
                    ws = _splat(wv, l)
                    for k in range(F // 16):
                        sl_ = pl.ds(k * 16, 16)
                        rb[e, sl_] = rb[e, sl_] * ws
                return carry

            lax.fori_loop(0, G16, gbody, 0)

        def chunk(j, *, peeled):
            u = j // 8

            if not peeled:
                @pl.when((j % 8 == 5) & (j < 8 * (NSUP - 1)))
                def _():
                    i_wait(u + 1, (u + 1) % 2)

            g_copy(j).wait()
            scale(j)
            s_issue(j)
            if not peeled:
                s_wait(j - 1)

                @pl.when(j + 3 <= NCHUNK - 1)
                def _():
                    g_copy(j + 3).start()

                @pl.when((j % 8 == 1) & (j >= 9) & (u <= NSUP - 2))
                def _():
                    i_issue(u + 1, (u + 1) % 2)
            else:
                g_copy(j + 3).start()

        i_issue(0, 0)
        i_wait(0, 0)
        i_issue(1, 1)
        g_copy(0).start()
        g_copy(1).start()
        g_copy(2).start()
        chunk(0, peeled=True)

        def jbody(j, carry):
            chunk(j, peeled=False)
            return carry

        lax.fori_loop(1, NCHUNK, jbody, 0)

        s_wait(NCHUNK - 1)

        plsc.subcore_barrier()
        for k5 in range(5):
            off = pl.multiple_of(r0 + k5 * 128, 8)

            @pl.when(off + 128 <= N)
            def _():
                pltpu.sync_copy(acc_sh.at[pl.ds(off, 128)],
                                acc_hbm.at[c, pl.ds(off, 128)])

        @pl.when(s == 15)
        def _():
            pltpu.sync_copy(acc_sh.at[pl.ds(N - 16, 16)],
                            acc_hbm.at[c, pl.ds(N - 16, 16)])

        if with_deg:
            @pl.when(s == 0)
            def _():
                pltpu.sync_copy(deg_sh, deg_hbm.at[c])

    return body


def _sc_pass(table, gidxr, sidxr, wr, F, with_deg):
    mesh = plsc.VectorSubcoreMesh(core_axis_name="c", subcore_axis_name="s")
    out_type = [jax.ShapeDtypeStruct((2, N, F), jnp.float32)]
    if with_deg:
        out_type.append(jax.ShapeDtypeStruct((2, N), jnp.float32))
    scratch = [
        pltpu.VMEM((2 * SCH,), jnp.int32),
        pltpu.VMEM((16, CH), jnp.int32),
        pltpu.VMEM((2 * SCH,), jnp.float32),
        pltpu.VMEM((4, CH, F), jnp.float32),
        pltpu.SemaphoreType.DMA((4,)),
        pltpu.SemaphoreType.DMA((4,)),
        pltpu.SemaphoreType.DMA((2,)),
        pltpu.VMEM_SHARED((N, F), jnp.float32),
    ]
    if with_deg:
        scratch.append(pltpu.VMEM_SHARED((N,), jnp.float32))
    f = pl.kernel(
        _mk_sc_body(F, with_deg),
        out_type=out_type,
        mesh=mesh,
        scratch_types=scratch,
        compiler_params=pltpu.CompilerParams(use_tc_tiling_on_sc=False),
    )
    args = [table, gidxr, sidxr, wr, jnp.zeros((128, F), jnp.float32)]
    if with_deg:
        args.append(jnp.zeros((N,), jnp.float32))
    out = f(*args)
    return out if with_deg else out[0]


def _tca_body(agg_ref, x_ref, wrel_ref, brel_ref, wroot_ref, wp_ref, bp_ref,
              h_ref, st_ref, sl_ref):
    agg = jnp.sum(agg_ref[...], axis=0)
    h = _mm(agg, wrel_ref[...]) + _mm(x_ref[...], wroot_ref[...]) + brel_ref[...]
    h = jnp.maximum(h, 0.0)
    sl = _mm(h, wp_ref[...]) + bp_ref[...]
    m = jnp.max(sl, axis=1, keepdims=True)
    e = jnp.exp(sl - m)
    s = e / jnp.sum(e, axis=1, keepdims=True)
    h_ref[...] = h
    sl_ref[...] = sl
    st_ref[...] = jnp.concatenate([s] * 8, axis=1)


def _tc_a(agg_parts, x, W_rel1, b_rel1, W_root1, Wp, bp):
    K = agg_parts.shape[0]
    return pl.pallas_call(
        _tca_body,
        grid=(NB,),
        in_specs=[
            pl.BlockSpec((K, BLK, F_IN), lambda i: (0, i, 0)),
            pl.BlockSpec((BLK, F_IN), lambda i: (i, 0)),
            pl.BlockSpec((F_IN, H), lambda i: (0, 0)),
            pl.BlockSpec((1, H), lambda i: (0, 0)),
            pl.BlockSpec((F_IN, H), lambda i: (0, 0)),
            pl.BlockSpec((H, C), lambda i: (0, 0)),
            pl.BlockSpec((1, C), lambda i: (0, 0)),
        ],
        out_specs=[
            pl.BlockSpec((BLK, H), lambda i: (i, 0)),
            pl.BlockSpec((BLK, 16), lambda i: (i, 0)),
            pl.BlockSpec((BLK, C), lambda i: (i, 0)),
        ],
        out_shape=[
            jax.ShapeDtypeStruct((N, H), jnp.float32),
            jax.ShapeDtypeStruct((N, 16), jnp.float32),
            jax.ShapeDtypeStruct((N, C), jnp.float32),
        ],
        interpret=_INTERPRET,
    )(agg_parts, x, W_rel1, b_rel1.reshape(1, H), W_root1, Wp, bp.reshape(1, C))


def _tcb_body(h_ref, st_ref, oh2_ref, q_ref, deg_ref, accO_ref, accM_ref):
    i = pl.program_id(0)
    P = oh2_ref[...] * st_ref[...]
    s = st_ref[...][:, 0:C]
    q = jnp.sum(q_ref[...], axis=0)[:, 0:C]
    deg = jnp.sum(deg_ref[...], axis=0)
    o16 = _dd0(P, h_ref[...])
    ss16 = _dd0(P, s)
    adj16 = _dd0(P, q)
    degssq = deg * jnp.sum(s * s, axis=1, keepdims=True)
    den16 = _dd0(oh2_ref[...], degssq)

    @pl.when(i == 0)
    def _():
        accO_ref[...] = jnp.zeros_like(accO_ref)
        accM_ref[...] = jnp.zeros_like(accM_ref)

    accO_ref[...] += o16
    accM_ref[:, 0:2] += ss16
    accM_ref[:, 2:4] += adj16
    accM_ref[:, 4:5] += den16


def _tc_b(h, st, oh2, q_parts, deg_parts):
    Kq = q_parts.shape[0]
    Kd = deg_parts.shape[0]
    return pl.pallas_call(
        _tcb_body,
        grid=(NB,),
        in_specs=[
            pl.BlockSpec((BLK, H), lambda i: (i, 0)),
            pl.BlockSpec((BLK, 16), lambda i: (i, 0)),
            pl.BlockSpec((BLK, 16), lambda i: (i, 0)),
            pl.BlockSpec((Kq, BLK, 16), lambda i: (0, i, 0)),
            pl.BlockSpec((Kd, BLK, 1), lambda i: (0, i, 0)),
        ],
        out_specs=[
            pl.BlockSpec((16, H), lambda i: (0, 0)),
            pl.BlockSpec((16, 128), lambda i: (0, 0)),
        ],
        out_shape=[
            jax.ShapeDtypeStruct((16, H), jnp.float32),
            jax.ShapeDtypeStruct((16, 128), jnp.float32),
        ],
        interpret=_INTERPRET,
    )(h, st, oh2, q_parts, deg_parts)


def _tcc_body(accO_ref, accM_ref, wr3_ref, br3_ref, wo3_ref, w1_ref, b1_ref,
              w2_ref, b2_ref, logp_ref, mc_ref, ol_ref, adj_ref):
    accO = accO_ref[...]
    accM = accM_ref[...]
    ss16 = accM[:, 0:2]
    adj16 = accM[:, 2:4]
    den16 = accM[:, 4:5]

    ri16 = jax.lax.broadcasted_iota(jnp.int32, (16, 16), 0)
    ci16 = jax.lax.broadcasted_iota(jnp.int32, (16, 16), 1)
    I16 = (ri16 == ci16).astype(jnp.float32)
    blockmask = (ri16 // 2 == ci16 // 2).astype(jnp.float32)
    G0 = (ci16 == 2 * (ri16 // 2)).astype(jnp.float32)
    G1 = (ci16 == 2 * (ri16 // 2) + 1).astype(jnp.float32)
    ri82 = jax.lax.broadcasted_iota(jnp.int32, (8, 16), 0)
    ci82 = jax.lax.broadcasted_iota(jnp.int32, (8, 16), 1)
    Pair8 = (ci82 // 2 == ri82).astype(jnp.float32)
    Geven = (ci82 == 2 * ri82).astype(jnp.float32)
    ri168 = jax.lax.broadcasted_iota(jnp.int32, (16, 8), 0)
    ci168 = jax.lax.broadcasted_iota(jnp.int32, (16, 8), 1)
    G2 = (ri168 // 2 == ci168).astype(jnp.float32)
    ri2 = jax.lax.broadcasted_iota(jnp.int32, (16, 2), 0)
    ci2 = jax.lax.broadcasted_iota(jnp.int32, (16, 2), 1)
    diagmask = (ci2 == ri2 % 2).astype(jnp.float32)

    diag16 = jnp.sum(adj16 * diagmask, axis=1, keepdims=True)
    num8 = _mm(Pair8, diag16)
    den8 = _mm(Geven, den16)
    mc = -jnp.mean(num8 / (den8 + EPS))
    mc_ref[...] = jnp.full((1, 1), 0.0) + mc

    rs = jnp.sum(ss16 * ss16, axis=1, keepdims=True)
    nb8 = jnp.sqrt(_mm(Pair8, rs))
    nb16 = _mm(G2, nb8)
    i_s16 = diagmask / math.sqrt(2.0)
    diff = ss16 / (nb16 + EPS) - i_s16
    fb8 = jnp.sqrt(_mm(Pair8, jnp.sum(diff * diff, axis=1, keepdims=True)))
    ol_ref[...] = jnp.full((1, 1), 0.0) + jnp.mean(fb8)

    adjz = adj16 * (1.0 - diagmask)
    d16 = jnp.sum(adjz, axis=1, keepdims=True)
    dsq = jnp.sqrt(d16 + EPS)
    dsp = jnp.concatenate([_mm(G0, dsq), _mm(G1, dsq)], axis=1)
    adjn = adjz / (dsq * dsp + EPS)
    adj_ref[...] = adjn

    adjnT = _dd0(adjn, I16)
    cond = (ri16 % 2 == 0)
    sel = jnp.where(cond, jnp.broadcast_to(adjnT[0:1, :], (16, 16)),
                    jnp.broadcast_to(adjnT[1:2, :], (16, 16)))
    M = blockmask * sel
    agg2 = _mm(M, accO)
    h2 = _mm(agg2, wr3_ref[...]) + br3_ref[...] + _mm(accO, wo3_ref[...])
    xg = 0.5 * _mm(Pair8, h2)
    xg = jnp.maximum(_mm(xg, w1_ref[...]) + b1_ref[...], 0.0)
    logits = _mm(xg, w2_ref[...]) + b2_ref[...]
    m = jnp.max(logits, axis=1, keepdims=True)
    lse = m + jnp.log(jnp.sum(jnp.exp(logits - m), axis=1, keepdims=True))
    logp_ref[...] = logits - lse


def _tc_c(accO, accM, W_rel3, b_rel3, W_root3, W1, b1, W2, b2):
    full = lambda shp: pl.BlockSpec(shp, lambda: tuple(0 for _ in shp))
    return pl.pallas_call(
        _tcc_body,
        grid=(),
        in_specs=[
            full((16, H)), full((16, 128)),
            full((H, H)), full((1, H)), full((H, H)),
            full((H, H)), full((1, H)), full((H, OUT)), full((1, OUT)),
        ],
        out_specs=[full((B, OUT)), full((1, 1)), full((1, 1)), full((16, C))],
        out_shape=[
            jax.ShapeDtypeStruct((B, OUT), jnp.float32),
            jax.ShapeDtypeStruct((1, 1), jnp.float32),
            jax.ShapeDtypeStruct((1, 1), jnp.float32),
            jax.ShapeDtypeStruct((16, C), jnp.float32),
        ],
        interpret=_INTERPRET,
    )(accO, accM, W_rel3, b_rel3.reshape(1, H), W_root3,
      W1, b1.reshape(1, H), W2, b2.reshape(1, OUT))


def kernel(x, edge_index, batch, edge_weight, W_rel1, b_rel1, W_root1, Wp, bp,
           W_rel3, b_rel3, W_root3, W1, b1, W2, b2):
    src = edge_index[0]
    dst = edge_index[1]
    pad = EPAD - E
    srcp = jnp.concatenate([src, jnp.zeros((pad,), src.dtype)])
    dstp = jnp.concatenate([dst, jnp.zeros((pad,), dst.dtype)])
    wr = jnp.concatenate([edge_weight,
                          jnp.zeros((pad,), jnp.float32)]).reshape(NW, EPT)
    agg, deg2 = _sc_pass(x, srcp.reshape(NW, EPT),
                         dstp.reshape(NW, NSUP, 8, CH), wr, F_IN, True)
    deg = deg2.reshape(2, N, 1)

    h, st, s_logits = _tc_a(agg, x, W_rel1, b_rel1, W_root1, Wp, bp)

    q16 = _sc_pass(st, dstp.reshape(NW, EPT),
                   srcp.reshape(NW, NSUP, 8, CH), wr, 16, False)

    oh2 = (batch[:, None] == (jnp.arange(16) // 2)).astype(jnp.float32)
    accO, accM = _tc_b(h, st, oh2, q16, deg)
    logp, mc, ol, adjn = _tc_c(accO, accM, W_rel3, b_rel3, W_root3,
                               W1, b1, W2, b2)
    return (logp, mc.reshape(()), ol.reshape(()), s_logits,
            adjn.reshape(B, C, C))


# speedup vs baseline: 42.1460x; 1.2906x over previous
import functools
import math

import jax
import jax.numpy as jnp
from jax import lax
from jax.experimental import pallas as pl
from jax.experimental.pallas import tpu as pltpu
from jax.experimental.pallas import tpu_sc as plsc

N = 10000
E = 320000
F_IN = 128
H = 512
C = 2
B = 8
OUT = 10
EPS = 1e-15
BLK = 1000
NB = N // BLK

NW = 32
CH = 48
SCH = 8 * CH
NS0 = 32
NS1 = 21
EPT0 = SCH * NS0
EPT1 = SCH * NS1
EPTROW = EPT0 + EPT1
EPAD = 16 * EPTROW
G16 = CH // 16

_INTERPRET = False


def _dd0(a, b):
    return jax.lax.dot_general(a, b, (((0,), (0,)), ((), ())),
                               preferred_element_type=jnp.float32)


def _mm(a, b):
    return jnp.dot(a, b, preferred_element_type=jnp.float32)


def _mk_sc_body(F, with_deg):
    def body(table_hbm, gidxr_hbm, sidxr_hbm, wr_hbm, zrows_hbm, *rest):
        if with_deg:
            (zn_hbm, acc_hbm, deg_hbm, gb, sb, wb, rows_v, gsem, ssem, isem,
             acc_sh, deg_sh) = rest
        else:
            acc_hbm, gb, sb, wb, rows_v, gsem, ssem, isem, acc_sh = rest
        c = lax.axis_index("c")
        s = lax.axis_index("s")
        ebase = pl.multiple_of(c * EPT0, 8)
        nsup = jnp.where(c == 0, NS0, NS1)
        nchunk = 8 * nsup

        r0 = pl.multiple_of(s * 640, 8)
        for k5 in range(5):
            off = pl.multiple_of(r0 + k5 * 128, 8)

            @pl.when(off + 128 <= N)
            def _():
                pltpu.sync_copy(zrows_hbm, acc_sh.at[pl.ds(off, 128)])

        @pl.when(s == 15)
        def _():
            pltpu.sync_copy(zrows_hbm.at[pl.ds(0, 16)],
                            acc_sh.at[pl.ds(N - 16, 16)])

        if with_deg:
            @pl.when(s == 0)
            def _():
                pltpu.sync_copy(zn_hbm, deg_sh)

        plsc.subcore_barrier()

        def i_copies(u, p):
            usl = pl.ds(pl.multiple_of(ebase + u * SCH, 8), SCH)
            psl = pl.ds(pl.multiple_of(p * SCH, 8), SCH)
            cps = [
                pltpu.make_async_copy(gidxr_hbm.at[s, usl], gb.at[psl],
                                      isem.at[p]),
                pltpu.make_async_copy(wr_hbm.at[s, usl], wb.at[psl],
                                      isem.at[p]),
            ]
            for r in range(8):
                rsl = pl.ds(pl.multiple_of(ebase + u * SCH + r * CH, 8), CH)
                cps.append(pltpu.make_async_copy(sidxr_hbm.at[s, rsl],
                                                 sb.at[p * 8 + r],
                                                 isem.at[p]))
            return cps

        def i_issue(u, p):
            for cp in i_copies(u, p):
                cp.start()

        def i_wait(u, p):
            for cp in i_copies(u, p):
                cp.wait()

        def _off(j):
            p, k8 = (j // 8) % 2, j % 8
            return p * SCH + k8 * CH

        def g_copy(j):
            b = j % 4
            return pltpu.make_async_copy(
                table_hbm.at[gb.at[pl.ds(pl.multiple_of(_off(j), 8), CH)]],
                rows_v.at[b], gsem.at[b])

        def _srow(j):
            p, k8 = (j // 8) % 2, j % 8
            return sb.at[p * 8 + k8]

        def s_issue(j):
            b = j % 4
            pltpu.async_copy(rows_v.at[b], acc_sh.at[_srow(j)], ssem.at[b],
                             add=True)
            if with_deg:
                pltpu.async_copy(
                    wb.at[pl.ds(pl.multiple_of(_off(j), 8), CH)],
                    deg_sh.at[_srow(j)], ssem.at[b], add=True)

        def s_wait(j):
            b = j % 4
            pltpu.make_async_copy(rows_v.at[b], acc_sh.at[_srow(j)],
                                  ssem.at[b]).wait()
            if with_deg:
                pltpu.make_async_copy(
                    wb.at[pl.ds(pl.multiple_of(_off(j), 8), CH)],
                    deg_sh.at[_srow(j)], ssem.at[b]).wait()

        dnums = lax.GatherDimensionNumbers(offset_dims=(),
                                           collapsed_slice_dims=(0,),
                                           start_index_map=(0,))

        def _splat(v16, l):
            idx = jnp.full((16, 1), l, jnp.int32)
            return lax.gather(v16, idx, dnums, (1,),
                              mode=lax.GatherScatterMode.PROMISE_IN_BOUNDS)

        def scale(j):
            b, p, k8 = j % 4, (j // 8) % 2, j % 8
            rb = rows_v.at[b]

            def gbody(g, carry):
                wv = wb[pl.ds(pl.multiple_of(p * SCH + k8 * CH + g * 16, 8),
                              16)]
                for l in range(16):
                    e = g * 16 + l
                    ws = _splat(wv, l)
                    for k in range(F // 16):
                        sl_ = pl.ds(k * 16, 16)
                        rb[e, sl_] = rb[e, sl_] * ws
                return carry

            lax.fori_loop(0, G16, gbody, 0)

        def chunk(j, *, peeled):
            u = j // 8

            if not peeled:
                @pl.when((j % 8 == 5) & (j < 8 * (nsup - 1)))
                def _():
                    i_wait(u + 1, (u + 1) % 2)

            g_copy(j).wait()
            scale(j)
            s_issue(j)
            if not peeled:
                s_wait(j - 1)

                @pl.when(j + 3 <= nchunk - 1)
                def _():
                    g_copy(j + 3).start()

                @pl.when((j % 8 == 1) & (j >= 9) & (u <= nsup - 2))
                def _():
                    i_issue(u + 1, (u + 1) % 2)
            else:
                g_copy(j + 3).start()

        i_issue(0, 0)
        i_wait(0, 0)
        i_issue(1, 1)
        g_copy(0).start()
        g_copy(1).start()
        g_copy(2).start()
        chunk(0, peeled=True)

        def jbody(j, carry):
            chunk(j, peeled=False)
            return carry

        lax.fori_loop(1, nchunk, jbody, 0)

        s_wait(nchunk - 1)

        plsc.subcore_barrier()
        for k5 in range(5):
            off = pl.multiple_of(r0 + k5 * 128, 8)

            @pl.when(off + 128 <= N)
            def _():
                pltpu.sync_copy(acc_sh.at[pl.ds(off, 128)],
                                acc_hbm.at[c, pl.ds(off, 128)])

        @pl.when(s == 15)
        def _():
            pltpu.sync_copy(acc_sh.at[pl.ds(N - 16, 16)],
                            acc_hbm.at[c, pl.ds(N - 16, 16)])

        if with_deg:
            @pl.when(s == 0)
            def _():
                pltpu.sync_copy(deg_sh, deg_hbm.at[c])

    return body


def _sc_pass(table, gidxr, sidxr, wr, F, with_deg):
    mesh = plsc.VectorSubcoreMesh(core_axis_name="c", subcore_axis_name="s")
    out_type = [jax.ShapeDtypeStruct((2, N, F), jnp.float32)]
    if with_deg:
        out_type.append(jax.ShapeDtypeStruct((2, N), jnp.float32))
    scratch = [
        pltpu.VMEM((2 * SCH,), jnp.int32),
        pltpu.VMEM((16, CH), jnp.int32),
        pltpu.VMEM((2 * SCH,), jnp.float32),
        pltpu.VMEM((4, CH, F), jnp.float32),
        pltpu.SemaphoreType.DMA((4,)),
        pltpu.SemaphoreType.DMA((4,)),
        pltpu.SemaphoreType.DMA((2,)),
        pltpu.VMEM_SHARED((N, F), jnp.float32),
    ]
    if with_deg:
        scratch.append(pltpu.VMEM_SHARED((N,), jnp.float32))
    f = pl.kernel(
        _mk_sc_body(F, with_deg),
        out_type=out_type,
        mesh=mesh,
        scratch_types=scratch,
        compiler_params=pltpu.CompilerParams(use_tc_tiling_on_sc=False),
    )
    args = [table, gidxr, sidxr, wr, jnp.zeros((128, F), jnp.float32)]
    if with_deg:
        args.append(jnp.zeros((N,), jnp.float32))
    out = f(*args)
    return out if with_deg else out[0]


def _tca_body(agg_ref, x_ref, wrel_ref, brel_ref, wroot_ref, wp_ref, bp_ref,
              h_ref, st_ref, sl_ref):
    agg = jnp.sum(agg_ref[...], axis=0)
    h = _mm(agg, wrel_ref[...]) + _mm(x_ref[...], wroot_ref[...]) + brel_ref[...]
    h = jnp.maximum(h, 0.0)
    sl = _mm(h, wp_ref[...]) + bp_ref[...]
    m = jnp.max(sl, axis=1, keepdims=True)
    e = jnp.exp(sl - m)
    s = e / jnp.sum(e, axis=1, keepdims=True)
    h_ref[...] = h
    sl_ref[...] = sl
    st_ref[...] = jnp.concatenate([s] * 8, axis=1)


def _tc_a(agg_parts, x, W_rel1, b_rel1, W_root1, Wp, bp):
    K = agg_parts.shape[0]
    return pl.pallas_call(
        _tca_body,
        grid=(NB,),
        in_specs=[
            pl.BlockSpec((K, BLK, F_IN), lambda i: (0, i, 0)),
            pl.BlockSpec((BLK, F_IN), lambda i: (i, 0)),
            pl.BlockSpec((F_IN, H), lambda i: (0, 0)),
            pl.BlockSpec((1, H), lambda i: (0, 0)),
            pl.BlockSpec((F_IN, H), lambda i: (0, 0)),
            pl.BlockSpec((H, C), lambda i: (0, 0)),
            pl.BlockSpec((1, C), lambda i: (0, 0)),
        ],
        out_specs=[
            pl.BlockSpec((BLK, H), lambda i: (i, 0)),
            pl.BlockSpec((BLK, 16), lambda i: (i, 0)),
            pl.BlockSpec((BLK, C), lambda i: (i, 0)),
        ],
        out_shape=[
            jax.ShapeDtypeStruct((N, H), jnp.float32),
            jax.ShapeDtypeStruct((N, 16), jnp.float32),
            jax.ShapeDtypeStruct((N, C), jnp.float32),
        ],
        interpret=_INTERPRET,
    )(agg_parts, x, W_rel1, b_rel1.reshape(1, H), W_root1, Wp, bp.reshape(1, C))


def _tcb_body(h_ref, st_ref, oh2_ref, q_ref, deg_ref, accO_ref, accM_ref):
    i = pl.program_id(0)
    P = oh2_ref[...] * st_ref[...]
    s = st_ref[...][:, 0:C]
    q = jnp.sum(q_ref[...], axis=0)[:, 0:C]
    deg = jnp.sum(deg_ref[...], axis=0)
    o16 = _dd0(P, h_ref[...])
    ss16 = _dd0(P, s)
    adj16 = _dd0(P, q)
    degssq = deg * jnp.sum(s * s, axis=1, keepdims=True)
    den16 = _dd0(oh2_ref[...], degssq)

    @pl.when(i == 0)
    def _():
        accO_ref[...] = jnp.zeros_like(accO_ref)
        accM_ref[...] = jnp.zeros_like(accM_ref)

    accO_ref[...] += o16
    accM_ref[:, 0:2] += ss16
    accM_ref[:, 2:4] += adj16
    accM_ref[:, 4:5] += den16


def _tc_b(h, st, oh2, q_parts, deg_parts):
    Kq = q_parts.shape[0]
    Kd = deg_parts.shape[0]
    return pl.pallas_call(
        _tcb_body,
        grid=(NB,),
        in_specs=[
            pl.BlockSpec((BLK, H), lambda i: (i, 0)),
            pl.BlockSpec((BLK, 16), lambda i: (i, 0)),
            pl.BlockSpec((BLK, 16), lambda i: (i, 0)),
            pl.BlockSpec((Kq, BLK, 16), lambda i: (0, i, 0)),
            pl.BlockSpec((Kd, BLK, 1), lambda i: (0, i, 0)),
        ],
        out_specs=[
            pl.BlockSpec((16, H), lambda i: (0, 0)),
            pl.BlockSpec((16, 128), lambda i: (0, 0)),
        ],
        out_shape=[
            jax.ShapeDtypeStruct((16, H), jnp.float32),
            jax.ShapeDtypeStruct((16, 128), jnp.float32),
        ],
        interpret=_INTERPRET,
    )(h, st, oh2, q_parts, deg_parts)


def _tcc_body(accO_ref, accM_ref, wr3_ref, br3_ref, wo3_ref, w1_ref, b1_ref,
              w2_ref, b2_ref, logp_ref, mc_ref, ol_ref, adj_ref):
    accO = accO_ref[...]
    accM = accM_ref[...]
    ss16 = accM[:, 0:2]
    adj16 = accM[:, 2:4]
    den16 = accM[:, 4:5]

    ri16 = jax.lax.broadcasted_iota(jnp.int32, (16, 16), 0)
    ci16 = jax.lax.broadcasted_iota(jnp.int32, (16, 16), 1)
    I16 = (ri16 == ci16).astype(jnp.float32)
    blockmask = (ri16 // 2 == ci16 // 2).astype(jnp.float32)
    G0 = (ci16 == 2 * (ri16 // 2)).astype(jnp.float32)
    G1 = (ci16 == 2 * (ri16 // 2) + 1).astype(jnp.float32)
    ri82 = jax.lax.broadcasted_iota(jnp.int32, (8, 16), 0)
    ci82 = jax.lax.broadcasted_iota(jnp.int32, (8, 16), 1)
    Pair8 = (ci82 // 2 == ri82).astype(jnp.float32)
    Geven = (ci82 == 2 * ri82).astype(jnp.float32)
    ri168 = jax.lax.broadcasted_iota(jnp.int32, (16, 8), 0)
    ci168 = jax.lax.broadcasted_iota(jnp.int32, (16, 8), 1)
    G2 = (ri168 // 2 == ci168).astype(jnp.float32)
    ri2 = jax.lax.broadcasted_iota(jnp.int32, (16, 2), 0)
    ci2 = jax.lax.broadcasted_iota(jnp.int32, (16, 2), 1)
    diagmask = (ci2 == ri2 % 2).astype(jnp.float32)

    diag16 = jnp.sum(adj16 * diagmask, axis=1, keepdims=True)
    num8 = _mm(Pair8, diag16)
    den8 = _mm(Geven, den16)
    mc = -jnp.mean(num8 / (den8 + EPS))
    mc_ref[...] = jnp.full((1, 1), 0.0) + mc

    rs = jnp.sum(ss16 * ss16, axis=1, keepdims=True)
    nb8 = jnp.sqrt(_mm(Pair8, rs))
    nb16 = _mm(G2, nb8)
    i_s16 = diagmask / math.sqrt(2.0)
    diff = ss16 / (nb16 + EPS) - i_s16
    fb8 = jnp.sqrt(_mm(Pair8, jnp.sum(diff * diff, axis=1, keepdims=True)))
    ol_ref[...] = jnp.full((1, 1), 0.0) + jnp.mean(fb8)

    adjz = adj16 * (1.0 - diagmask)
    d16 = jnp.sum(adjz, axis=1, keepdims=True)
    dsq = jnp.sqrt(d16 + EPS)
    dsp = jnp.concatenate([_mm(G0, dsq), _mm(G1, dsq)], axis=1)
    adjn = adjz / (dsq * dsp + EPS)
    adj_ref[...] = adjn

    adjnT = _dd0(adjn, I16)
    cond = (ri16 % 2 == 0)
    sel = jnp.where(cond, jnp.broadcast_to(adjnT[0:1, :], (16, 16)),
                    jnp.broadcast_to(adjnT[1:2, :], (16, 16)))
    M = blockmask * sel
    agg2 = _mm(M, accO)
    h2 = _mm(agg2, wr3_ref[...]) + br3_ref[...] + _mm(accO, wo3_ref[...])
    xg = 0.5 * _mm(Pair8, h2)
    xg = jnp.maximum(_mm(xg, w1_ref[...]) + b1_ref[...], 0.0)
    logits = _mm(xg, w2_ref[...]) + b2_ref[...]
    m = jnp.max(logits, axis=1, keepdims=True)
    lse = m + jnp.log(jnp.sum(jnp.exp(logits - m), axis=1, keepdims=True))
    logp_ref[...] = logits - lse


def _tc_c(accO, accM, W_rel3, b_rel3, W_root3, W1, b1, W2, b2):
    full = lambda shp: pl.BlockSpec(shp, lambda: tuple(0 for _ in shp))
    return pl.pallas_call(
        _tcc_body,
        grid=(),
        in_specs=[
            full((16, H)), full((16, 128)),
            full((H, H)), full((1, H)), full((H, H)),
            full((H, H)), full((1, H)), full((H, OUT)), full((1, OUT)),
        ],
        out_specs=[full((B, OUT)), full((1, 1)), full((1, 1)), full((16, C))],
        out_shape=[
            jax.ShapeDtypeStruct((B, OUT), jnp.float32),
            jax.ShapeDtypeStruct((1, 1), jnp.float32),
            jax.ShapeDtypeStruct((1, 1), jnp.float32),
            jax.ShapeDtypeStruct((16, C), jnp.float32),
        ],
        interpret=_INTERPRET,
    )(accO, accM, W_rel3, b_rel3.reshape(1, H), W_root3,
      W1, b1.reshape(1, H), W2, b2.reshape(1, OUT))


def kernel(x, edge_index, batch, edge_weight, W_rel1, b_rel1, W_root1, Wp, bp,
           W_rel3, b_rel3, W_root3, W1, b1, W2, b2):
    src = edge_index[0]
    dst = edge_index[1]
    pad = EPAD - E
    srcp = jnp.concatenate([src, jnp.zeros((pad,), src.dtype)]).reshape(
        16, EPTROW)
    dstp = jnp.concatenate([dst, jnp.zeros((pad,), dst.dtype)]).reshape(
        16, EPTROW)
    wr = jnp.concatenate([edge_weight,
                          jnp.zeros((pad,), jnp.float32)]).reshape(16, EPTROW)
    agg, deg2 = _sc_pass(x, srcp, dstp, wr, F_IN, True)
    deg = deg2.reshape(2, N, 1)

    h, st, s_logits = _tc_a(agg, x, W_rel1, b_rel1, W_root1, Wp, bp)

    q16 = _sc_pass(st, dstp, srcp, wr, 16, False)

    oh2 = (batch[:, None] == (jnp.arange(16) // 2)).astype(jnp.float32)
    accO, accM = _tc_b(h, st, oh2, q16, deg)
    logp, mc, ol, adjn = _tc_c(accO, accM, W_rel3, b_rel3, W_root3,
                               W1, b1, W2, b2)
    return (logp, mc.reshape(()), ol.reshape(()), s_logits,
            adjn.reshape(B, C, C))
